# baseline (speedup 1.0000x reference)
import functools

import jax
import jax.numpy as jnp
from jax import lax
from jax.experimental import pallas as pl
from jax.experimental.pallas import tpu as pltpu
from jax.experimental.pallas import tpu_sc as plsc

_N = 10000
_E = 160000
_D = 256
_NS = 16
_G = 4
_DG = _D // _G
_ET = _E // _NS
_CH = 80
_NCHUNK = _ET // _CH
_NSLOT = 2
_FS = _NCHUNK // _NSLOT
_LEFT = _NCHUNK - _NSLOT * _FS
_NPS = _N // _NS

_TN = 1024
_TPAD = 1040
_TSCALE = _TN / 36.0
_TMAX = 1023.999


def _prep_body(Vw, Cw, Aw, Bw, Dw, Ew, Vb, Ab, Bb, Cb, Db, Eb, Wall, ball):
    Vwv = Vw[...]
    Cwv = Cw[...]
    Ws = Dw[...] + 2.0 * jnp.dot(Vwv[0:_D, :], Cwv,
                                 preferred_element_type=jnp.float32)
    Wt = Ew[...] + 2.0 * jnp.dot(Vwv[_D:2 * _D, :], Cwv,
                                 preferred_element_type=jnp.float32)
    cs = Db[...] + Cb[...] + 2.0 * jnp.dot(Vb[...], Cwv,
                                           preferred_element_type=jnp.float32)
    Bwv = Bw[...]
    Bbv = Bb[...]
    Wall[:, 0:_D] = Aw[...]
    ball[:, 0:_D] = Ab[...]
    for g in range(_G):
        base = _D + g * 2 * _DG
        Wall[:, base:base + _DG] = Ws[:, g * _DG:(g + 1) * _DG]
        Wall[:, base + _DG:base + 2 * _DG] = Bwv[:, g * _DG:(g + 1) * _DG]
        ball[:, base:base + _DG] = cs[:, g * _DG:(g + 1) * _DG]
        ball[:, base + _DG:base + 2 * _DG] = Bbv[:, g * _DG:(g + 1) * _DG]
    Wall[:, 3 * _D:4 * _D] = Wt
    ball[:, 3 * _D:4 * _D] = Eb[...]


def _prep_weights(V_w, C_w, A_w, B_w, D_w, E_w, V_b, A_b, B_b, C_b, D_b, E_b):
    full = lambda shp: pl.BlockSpec(shp, lambda: (0,) * len(shp))
    return pl.pallas_call(
        _prep_body,
        out_shape=[
            jax.ShapeDtypeStruct((_D, 4 * _D), jnp.float32),
            jax.ShapeDtypeStruct((1, 4 * _D), jnp.float32),
        ],
        in_specs=[full((2 * _D, _D))] + [full((_D, _D))] * 5
        + [full((1, _D))] * 6,
        out_specs=[full((_D, 4 * _D)), full((1, 4 * _D))],
    )(V_w, C_w, A_w, B_w, D_w, E_w, V_b, A_b, B_b, C_b, D_b, E_b)


_NB = 1000


def _node_mm_body(h, Ww, Wb, Wall, ball, ah, sb, t):
    hw = jnp.dot(h[...], Ww[...], preferred_element_type=jnp.float32) + Wb[...]
    z = jnp.dot(hw, Wall[...], preferred_element_type=jnp.float32) + ball[...]
    ah[...] = z[:, 0:_D]
    for g in range(_G):
        sb[g] = z[:, _D + g * 2 * _DG:_D + (g + 1) * 2 * _DG]
        t[g] = z[:, 3 * _D + g * _DG:3 * _D + (g + 1) * _DG]


def _node_matmuls(h, W_w, W_b, Wall, ball):
    return pl.pallas_call(
        _node_mm_body,
        grid=(_N // _NB,),
        out_shape=[
            jax.ShapeDtypeStruct((_N, _D), jnp.float32),
            jax.ShapeDtypeStruct((_G, _N, 2 * _DG), jnp.float32),
            jax.ShapeDtypeStruct((_G, _N, _DG), jnp.float32),
        ],
        in_specs=[
            pl.BlockSpec((_NB, _D), lambda i: (i, 0)),
            pl.BlockSpec((_D, _D), lambda i: (0, 0)),
            pl.BlockSpec((1, _D), lambda i: (0, 0)),
            pl.BlockSpec((_D, 4 * _D), lambda i: (0, 0)),
            pl.BlockSpec((1, 4 * _D), lambda i: (0, 0)),
        ],
        out_specs=[
            pl.BlockSpec((_NB, _D), lambda i: (i, 0)),
            pl.BlockSpec((_G, _NB, 2 * _DG), lambda i: (0, i, 0)),
            pl.BlockSpec((_G, _NB, _DG), lambda i: (0, i, 0)),
        ],
    )(h, W_w, W_b, Wall, ball)


@functools.cache
def _build_edge_kernel():
    mesh = plsc.VectorSubcoreMesh(core_axis_name="c", subcore_axis_name="s",
                                  num_cores=2, num_subcores=_NS)
    return functools.partial(
        pl.kernel,
        mesh=mesh,
        compiler_params=pltpu.CompilerParams(use_tc_tiling_on_sc=False,
                                             needs_layout_passes=False),
        out_type=[
        jax.ShapeDtypeStruct((_E, _D), jnp.float32),
        jax.ShapeDtypeStruct((_N, _D), jnp.float32),
        jax.ShapeDtypeStruct((_N, _D), jnp.float32),
        jax.ShapeDtypeStruct((_G, 2 * _NS, _DG), jnp.float32),
    ],
        scratch_types=[
        [pltpu.VMEM((2, _CH), jnp.int32)] * _NSLOT,
        [pltpu.VMEM((_CH, 2 * _DG), jnp.float32)] * _NSLOT,
        [pltpu.VMEM((_CH, _DG), jnp.float32)] * _NSLOT,
        pltpu.VMEM((_CH, _DG), jnp.float32),
        pltpu.VMEM((_CH, 2 * _DG), jnp.float32),
        pltpu.VMEM((2, _DG), jnp.float32),
        pltpu.VMEM((_TPAD,), jnp.float32),
        pltpu.VMEM_SHARED((_N, 2 * _DG), jnp.float32),
        [pltpu.SemaphoreType.DMA] * _NSLOT,
        [pltpu.SemaphoreType.DMA] * _NSLOT,
        pltpu.SemaphoreType.DMA,
        [pltpu.SemaphoreType.DMA] * _NSLOT,
    ],
    )(_edge_body)


def _edge_call(*args):
    return _build_edge_kernel()(*args)


def _edge_body(sb0, sb1, sb2, sb3, t0, t1, t2, t3, ei, zrows,
               e_out, ssh_out, ss_out, stats_out,
               idx_v, sb_v, t_v, e_v, psg_v, stats_v, tbl_v,
               accC, gsem_sb, gsem_t, esem, isem):
    c = lax.axis_index("c")
    s = lax.axis_index("s")

    def fill_tbl(j, zero):
        idx = j * 16 + lax.iota(jnp.int32, 16)
        x = (idx.astype(jnp.float32) - 512.0) * (1.0 / _TSCALE)
        tbl_v[pl.ds(j * 16, 16)] = 1.0 / (1.0 + jnp.exp(-x))
        return zero

    lax.fori_loop(0, _TPAD // 16, fill_tbl, 0)

    def process(g, sbh, th):
        row0 = s * _NPS
        pltpu.sync_copy(zrows.at[pl.ds(row0, _NPS)], accC.at[pl.ds(row0, _NPS)])
        plsc.subcore_barrier()
        ebase = s * _ET
        ecol = pl.ds(g * _DG, _DG)

        for b in range(_NSLOT):
            eb0 = ebase + b * _CH
            pltpu.sync_copy(ei.at[:, pl.ds(eb0, _CH)], idx_v[b])
            pltpu.async_copy(sbh.at[idx_v[b].at[0]], sb_v[b], gsem_sb[b])
            pltpu.async_copy(th.at[idx_v[b].at[1]], t_v[b], gsem_t[b])

        def chunk_work(b, eb, ebn, carry, i, pf):
            pltpu.make_async_copy(sbh.at[idx_v[b].at[0]], sb_v[b],
                                  gsem_sb[b]).wait()
            pltpu.make_async_copy(th.at[idx_v[b].at[1]], t_v[b],
                                  gsem_t[b]).wait()

            if b == 0:
                @pl.when(i > 0)
                def _():
                    pltpu.make_async_copy(
                        e_v, e_out.at[pl.ds(eb, _CH), ecol], esem).wait()
            else:
                pltpu.make_async_copy(
                    e_v, e_out.at[pl.ds(eb, _CH), ecol], esem).wait()

            def row_body(r, rc):
                nc = _DG // 16
                sls = [pl.ds(cc * 16, 16) for cc in range(nc)]
                bsls = [pl.ds(_DG + cc * 16, 16) for cc in range(nc)]
                svs = [sb_v[b][r, sl] for sl in sls]
                tvs = [t_v[b][r, sl] for sl in sls]
                bvs = [sb_v[b][r, sl] for sl in bsls]
                es = [sv + tv for sv, tv in zip(svs, tvs)]
                us = [jnp.minimum(jnp.maximum(e * _TSCALE + 512.0, 0.0),
                                  _TMAX) for e in es]
                i0s = [u.astype(jnp.int32) for u in us]
                fs = [u - i0.astype(jnp.float32)
                      for u, i0 in zip(us, i0s)]
                v0s = [plsc.load_gather(tbl_v, [i0]) for i0 in i0s]
                v1s = [plsc.load_gather(tbl_v, [i0 + 1]) for i0 in i0s]
                sigs = [v0 + f * (v1 - v0)
                        for v0, v1, f in zip(v0s, v1s, fs)]
                for cc in range(nc):
                    e_v[r, sls[cc]] = es[cc]
                    psg_v[r, bsls[cc]] = sigs[cc]
                    psg_v[r, sls[cc]] = bvs[cc] * sigs[cc]
                sums = [rc[cc] + es[cc] for cc in range(nc)]
                sqs = [rc[4 + cc] + es[cc] * es[cc] for cc in range(nc)]
                return tuple(sums + sqs)

            carry = plsc.parallel_loop(0, _CH, carry=carry)(row_body)
            pltpu.sync_copy(psg_v, accC.at[idx_v[b].at[1]], add=True)
            pltpu.async_copy(e_v, e_out.at[pl.ds(eb, _CH), ecol], esem)

            if pf and b < _LEFT:
                pltpu.async_copy(ei.at[:, pl.ds(ebn, _CH)], idx_v[b], isem[b])
                pltpu.make_async_copy(ei.at[:, pl.ds(ebn, _CH)], idx_v[b],
                                      isem[b]).wait()
                pltpu.async_copy(sbh.at[idx_v[b].at[0]], sb_v[b], gsem_sb[b])
                pltpu.async_copy(th.at[idx_v[b].at[1]], t_v[b], gsem_t[b])
            elif pf:
                @pl.when(i < _FS - 1)
                def _():
                    pltpu.async_copy(ei.at[:, pl.ds(ebn, _CH)], idx_v[b],
                                     isem[b])
                    pltpu.make_async_copy(ei.at[:, pl.ds(ebn, _CH)], idx_v[b],
                                          isem[b]).wait()
                    pltpu.async_copy(sbh.at[idx_v[b].at[0]], sb_v[b],
                                     gsem_sb[b])
                    pltpu.async_copy(th.at[idx_v[b].at[1]], t_v[b], gsem_t[b])
            return carry

        def step(i, carry):
            for b in range(_NSLOT):
                eb = ebase + (_NSLOT * i + b) * _CH
                ebn = eb + _NSLOT * _CH
                carry = chunk_work(b, eb, ebn, carry, i, True)
            return carry

        z16 = jnp.zeros((16,), jnp.float32)
        res = lax.fori_loop(0, _FS, step, (z16,) * 8)
        for b in range(_LEFT):
            eb = ebase + (_NSLOT * _FS + b) * _CH
            res = chunk_work(b, eb, eb, res, jnp.int32(_FS), False)
        pltpu.make_async_copy(
            e_v, e_out.at[pl.ds(ebase, _CH), ecol], esem).wait()
        for cc in range(_DG // 16):
            stats_v[0, pl.ds(cc * 16, 16)] = res[cc]
            stats_v[1, pl.ds(cc * 16, 16)] = res[4 + cc]
        pltpu.sync_copy(stats_v, stats_out.at[g, pl.ds(2 * s, 2)])
        plsc.subcore_barrier()
        pltpu.sync_copy(accC.at[pl.ds(row0, _NPS), pl.ds(0, _DG)],
                        ssh_out.at[pl.ds(row0, _NPS), ecol])
        pltpu.sync_copy(accC.at[pl.ds(row0, _NPS), pl.ds(_DG, _DG)],
                        ss_out.at[pl.ds(row0, _NPS), ecol])
        plsc.subcore_barrier()

    @pl.when(c == 0)
    def _():
        process(0, sb0, t0)
        process(1, sb1, t1)

    @pl.when(c == 1)
    def _():
        process(2, sb2, t2)
        process(3, sb3, t3)


_EB = 2000


def _enorm_body(e, scale, shift, out):
    out[...] = jnp.maximum(e[...] * scale[...] + shift[...], 0.0)


def _e_norm(e_raw, scale, shift):
    return pl.pallas_call(
        _enorm_body,
        grid=(_E // _EB,),
        out_shape=jax.ShapeDtypeStruct((_E, _D), jnp.float32),
        in_specs=[
            pl.BlockSpec((_EB, _D), lambda i: (i, 0)),
            pl.BlockSpec((1, _D), lambda i: (0, 0)),
            pl.BlockSpec((1, _D), lambda i: (0, 0)),
        ],
        out_specs=pl.BlockSpec((_EB, _D), lambda i: (i, 0)),
    )(e_raw, scale, shift)


def _node_update_body(ah, ssh, ss, sn, hn, sums):
    i = pl.program_id(0)
    h = (ah[...] + ssh[...] / (ss[...] + 1e-6)) * sn[...]
    hn[...] = h
    s1 = jnp.sum(h, axis=0)[None, :]
    s2 = jnp.sum(h * h, axis=0)[None, :]
    blk = jnp.concatenate([s1, s2, jnp.zeros((6, _D), jnp.float32)], axis=0)

    @pl.when(i == 0)
    def _():
        sums[...] = blk

    @pl.when(i > 0)
    def _():
        sums[...] += blk


def _node_update(Ah, ssh, ss, snorm_n):
    return pl.pallas_call(
        _node_update_body,
        grid=(_N // _NB,),
        out_shape=[
            jax.ShapeDtypeStruct((_N, _D), jnp.float32),
            jax.ShapeDtypeStruct((8, _D), jnp.float32),
        ],
        in_specs=[
            pl.BlockSpec((_NB, _D), lambda i: (i, 0)),
            pl.BlockSpec((_NB, _D), lambda i: (i, 0)),
            pl.BlockSpec((_NB, _D), lambda i: (i, 0)),
            pl.BlockSpec((_NB, 1), lambda i: (i, 0)),
        ],
        out_specs=[
            pl.BlockSpec((_NB, _D), lambda i: (i, 0)),
            pl.BlockSpec((8, _D), lambda i: (0, 0)),
        ],
    )(Ah, ssh, ss, snorm_n)


def _hnorm_body(h, scale, shift, out):
    out[...] = jnp.maximum(h[...] * scale[...] + shift[...], 0.0)


def _h_norm(hn_pre, scale, shift):
    return pl.pallas_call(
        _hnorm_body,
        grid=(_N // _NB,),
        out_shape=jax.ShapeDtypeStruct((_N, _D), jnp.float32),
        in_specs=[
            pl.BlockSpec((_NB, _D), lambda i: (i, 0)),
            pl.BlockSpec((1, _D), lambda i: (0, 0)),
            pl.BlockSpec((1, _D), lambda i: (0, 0)),
        ],
        out_specs=pl.BlockSpec((_NB, _D), lambda i: (i, 0)),
    )(hn_pre, scale, shift)


def kernel(h, edge_index, snorm_n, W_w, W_b, V_w, V_b, A_w, A_b, B_w, B_b,
           C_w, C_b, D_w, D_b, E_w, E_b, gamma_h, beta_h, gamma_e, beta_e):
    r2 = lambda v: v.reshape(1, _D)
    Wall, ball = _prep_weights(V_w, C_w, A_w, B_w, D_w, E_w, r2(V_b),
                               r2(A_b), r2(B_b), r2(C_b), r2(D_b), r2(E_b))
    Ah, sb4, t4 = _node_matmuls(h, W_w, r2(W_b), Wall, ball)

    zrows = jnp.zeros((_N, 2 * _DG), jnp.float32)
    e_raw, ssh, ss, stats = _edge_call(
        sb4[0], sb4[1], sb4[2], sb4[3], t4[0], t4[1], t4[2], t4[3],
        edge_index, zrows)

    sum_e = jnp.sum(stats[:, 0::2, :], axis=1).reshape(_D)
    sumsq_e = jnp.sum(stats[:, 1::2, :], axis=1).reshape(_D)
    mean_e = sum_e / _E
    var_e = sumsq_e / _E - mean_e * mean_e
    scale_e = gamma_e / jnp.sqrt(var_e + 1e-5)
    shift_e = beta_e - mean_e * scale_e
    e_out = _e_norm(e_raw, r2(scale_e), r2(shift_e))

    hn_pre, sums = _node_update(Ah, ssh, ss, snorm_n)
    mean_h = sums[0] / _N
    var_h = sums[1] / _N - mean_h * mean_h
    scale_h = gamma_h / jnp.sqrt(var_h + 1e-5)
    shift_h = beta_h - mean_h * scale_h
    hn = _h_norm(hn_pre, r2(scale_h), r2(shift_h))
    return (hn, e_out)


# speedup vs baseline: 1.0028x; 1.0028x over previous
import functools

import jax
import jax.numpy as jnp
from jax import lax
from jax.experimental import pallas as pl
from jax.experimental.pallas import tpu as pltpu
from jax.experimental.pallas import tpu_sc as plsc

_N = 10000
_E = 160000
_D = 256
_NS = 16
_G = 4
_DG = _D // _G
_ET = _E // _NS
_CH = 80
_NCHUNK = _ET // _CH
_NSLOT = 2
_FS = _NCHUNK // _NSLOT
_LEFT = _NCHUNK - _NSLOT * _FS
_NPS = _N // _NS

_TN = 1024
_TPAD = 1040
_TSCALE = _TN / 36.0
_TMAX = 1023.999


def _prep_body(Vw, Cw, Aw, Bw, Dw, Ew, Vb, Ab, Bb, Cb, Db, Eb, Wall, ball):
    Vwv = Vw[...]
    Cwv = Cw[...]
    Ws = Dw[...] + 2.0 * jnp.dot(Vwv[0:_D, :], Cwv,
                                 preferred_element_type=jnp.float32)
    Wt = Ew[...] + 2.0 * jnp.dot(Vwv[_D:2 * _D, :], Cwv,
                                 preferred_element_type=jnp.float32)
    cs = Db[...] + Cb[...] + 2.0 * jnp.dot(Vb[...], Cwv,
                                           preferred_element_type=jnp.float32)
    Bwv = Bw[...]
    Bbv = Bb[...]
    Wall[:, 0:_D] = Aw[...]
    ball[:, 0:_D] = Ab[...]
    for g in range(_G):
        base = _D + g * 2 * _DG
        Wall[:, base:base + _DG] = Ws[:, g * _DG:(g + 1) * _DG]
        Wall[:, base + _DG:base + 2 * _DG] = Bwv[:, g * _DG:(g + 1) * _DG]
        ball[:, base:base + _DG] = cs[:, g * _DG:(g + 1) * _DG]
        ball[:, base + _DG:base + 2 * _DG] = Bbv[:, g * _DG:(g + 1) * _DG]
    Wall[:, 3 * _D:4 * _D] = Wt
    ball[:, 3 * _D:4 * _D] = Eb[...]


def _prep_weights(V_w, C_w, A_w, B_w, D_w, E_w, V_b, A_b, B_b, C_b, D_b, E_b):
    full = lambda shp: pl.BlockSpec(shp, lambda: (0,) * len(shp))
    return pl.pallas_call(
        _prep_body,
        out_shape=[
            jax.ShapeDtypeStruct((_D, 4 * _D), jnp.float32),
            jax.ShapeDtypeStruct((1, 4 * _D), jnp.float32),
        ],
        in_specs=[full((2 * _D, _D))] + [full((_D, _D))] * 5
        + [full((1, _D))] * 6,
        out_specs=[full((_D, 4 * _D)), full((1, 4 * _D))],
    )(V_w, C_w, A_w, B_w, D_w, E_w, V_b, A_b, B_b, C_b, D_b, E_b)


_NB = 1000


def _node_mm_body(h, Ww, Wb, Wall, ball, ah, sb, t):
    hw = jnp.dot(h[...], Ww[...], preferred_element_type=jnp.float32) + Wb[...]
    z = jnp.dot(hw, Wall[...], preferred_element_type=jnp.float32) + ball[...]
    ah[...] = z[:, 0:_D]
    for g in range(_G):
        sb[g] = z[:, _D + g * 2 * _DG:_D + (g + 1) * 2 * _DG]
        t[g] = z[:, 3 * _D + g * _DG:3 * _D + (g + 1) * _DG]


def _node_matmuls(h, W_w, W_b, Wall, ball):
    return pl.pallas_call(
        _node_mm_body,
        grid=(_N // _NB,),
        out_shape=[
            jax.ShapeDtypeStruct((_N, _D), jnp.float32),
            jax.ShapeDtypeStruct((_G, _N, 2 * _DG), jnp.float32),
            jax.ShapeDtypeStruct((_G, _N, _DG), jnp.float32),
        ],
        in_specs=[
            pl.BlockSpec((_NB, _D), lambda i: (i, 0)),
            pl.BlockSpec((_D, _D), lambda i: (0, 0)),
            pl.BlockSpec((1, _D), lambda i: (0, 0)),
            pl.BlockSpec((_D, 4 * _D), lambda i: (0, 0)),
            pl.BlockSpec((1, 4 * _D), lambda i: (0, 0)),
        ],
        out_specs=[
            pl.BlockSpec((_NB, _D), lambda i: (i, 0)),
            pl.BlockSpec((_G, _NB, 2 * _DG), lambda i: (0, i, 0)),
            pl.BlockSpec((_G, _NB, _DG), lambda i: (0, i, 0)),
        ],
    )(h, W_w, W_b, Wall, ball)


@functools.cache
def _build_edge_kernel():
    mesh = plsc.VectorSubcoreMesh(core_axis_name="c", subcore_axis_name="s",
                                  num_cores=2, num_subcores=_NS)
    return functools.partial(
        pl.kernel,
        mesh=mesh,
        compiler_params=pltpu.CompilerParams(use_tc_tiling_on_sc=False,
                                             needs_layout_passes=False),
        out_type=[
        jax.ShapeDtypeStruct((_E, _D), jnp.float32),
        jax.ShapeDtypeStruct((_N, _D), jnp.float32),
        jax.ShapeDtypeStruct((_N, _D), jnp.float32),
        jax.ShapeDtypeStruct((_G, 2 * _NS, _DG), jnp.float32),
    ],
        scratch_types=[
        [pltpu.VMEM((_CH,), jnp.int32)] * _NSLOT,
        [pltpu.VMEM((_CH,), jnp.int32)] * _NSLOT,
        [pltpu.VMEM((_CH, 2 * _DG), jnp.float32)] * _NSLOT,
        [pltpu.VMEM((_CH, _DG), jnp.float32)] * _NSLOT,
        pltpu.VMEM((_CH, _DG), jnp.float32),
        pltpu.VMEM((_CH, 2 * _DG), jnp.float32),
        pltpu.VMEM((2, _DG), jnp.float32),
        pltpu.VMEM((_TPAD,), jnp.float32),
        pltpu.VMEM_SHARED((_N, 2 * _DG), jnp.float32),
        [pltpu.SemaphoreType.DMA] * _NSLOT,
        [pltpu.SemaphoreType.DMA] * _NSLOT,
        pltpu.SemaphoreType.DMA,
        [pltpu.SemaphoreType.DMA] * _NSLOT,
    ],
    )(_edge_body)


def _edge_call(*args):
    return _build_edge_kernel()(*args)


def _edge_body(sb0, sb1, sb2, sb3, t0, t1, t2, t3, src, dst, zrows,
               e_out, ssh_out, ss_out, stats_out,
               src_v, dst_v, sb_v, t_v, e_v, psg_v, stats_v, tbl_v,
               accC, gsem_sb, gsem_t, esem, isem):
    c = lax.axis_index("c")
    s = lax.axis_index("s")

    def fill_tbl(j, zero):
        idx = j * 16 + lax.iota(jnp.int32, 16)
        x = (idx.astype(jnp.float32) - 512.0) * (1.0 / _TSCALE)
        tbl_v[pl.ds(j * 16, 16)] = 1.0 / (1.0 + jnp.exp(-x))
        return zero

    lax.fori_loop(0, _TPAD // 16, fill_tbl, 0)

    def process(g, sbh, th):
        row0 = s * _NPS
        pltpu.sync_copy(zrows.at[pl.ds(row0, _NPS)], accC.at[pl.ds(row0, _NPS)])
        plsc.subcore_barrier()
        ebase = s * _ET
        ecol = pl.ds(g * _DG, _DG)

        for b in range(_NSLOT):
            eb0 = ebase + b * _CH
            pltpu.sync_copy(src.at[pl.ds(eb0, _CH)], src_v[b])
            pltpu.sync_copy(dst.at[pl.ds(eb0, _CH)], dst_v[b])
            pltpu.async_copy(sbh.at[src_v[b]], sb_v[b], gsem_sb[b])
            pltpu.async_copy(th.at[dst_v[b]], t_v[b], gsem_t[b])

        def chunk_work(b, eb, ebn, carry, i, pf):
            pltpu.make_async_copy(sbh.at[src_v[b]], sb_v[b],
                                  gsem_sb[b]).wait()
            pltpu.make_async_copy(th.at[dst_v[b]], t_v[b],
                                  gsem_t[b]).wait()

            if b == 0:
                @pl.when(i > 0)
                def _():
                    pltpu.make_async_copy(
                        e_v, e_out.at[pl.ds(eb, _CH), ecol], esem).wait()
            else:
                pltpu.make_async_copy(
                    e_v, e_out.at[pl.ds(eb, _CH), ecol], esem).wait()

            if pf and b < _LEFT:
                pltpu.async_copy(src.at[pl.ds(ebn, _CH)], src_v[b], isem[b])
            elif pf:
                @pl.when(i < _FS - 1)
                def _():
                    pltpu.async_copy(src.at[pl.ds(ebn, _CH)], src_v[b],
                                     isem[b])

            def row_body(r, rc):
                nc = _DG // 16
                sls = [pl.ds(cc * 16, 16) for cc in range(nc)]
                bsls = [pl.ds(_DG + cc * 16, 16) for cc in range(nc)]
                svs = [sb_v[b][r, sl] for sl in sls]
                tvs = [t_v[b][r, sl] for sl in sls]
                bvs = [sb_v[b][r, sl] for sl in bsls]
                es = [sv + tv for sv, tv in zip(svs, tvs)]
                us = [jnp.minimum(jnp.maximum(e * _TSCALE + 512.0, 0.0),
                                  _TMAX) for e in es]
                i0s = [u.astype(jnp.int32) for u in us]
                fs = [u - i0.astype(jnp.float32)
                      for u, i0 in zip(us, i0s)]
                v0s = [plsc.load_gather(tbl_v, [i0]) for i0 in i0s]
                v1s = [plsc.load_gather(tbl_v, [i0 + 1]) for i0 in i0s]
                sigs = [v0 + f * (v1 - v0)
                        for v0, v1, f in zip(v0s, v1s, fs)]
                for cc in range(nc):
                    e_v[r, sls[cc]] = es[cc]
                    psg_v[r, bsls[cc]] = sigs[cc]
                    psg_v[r, sls[cc]] = bvs[cc] * sigs[cc]
                sums = [rc[cc] + es[cc] for cc in range(nc)]
                sqs = [rc[4 + cc] + es[cc] * es[cc] for cc in range(nc)]
                return tuple(sums + sqs)

            carry = plsc.parallel_loop(0, _CH, carry=carry)(row_body)
            pltpu.sync_copy(psg_v, accC.at[dst_v[b]], add=True)
            pltpu.async_copy(e_v, e_out.at[pl.ds(eb, _CH), ecol], esem)

            if pf and b < _LEFT:
                pltpu.async_copy(dst.at[pl.ds(ebn, _CH)], dst_v[b], isem[b])
                pltpu.make_async_copy(src.at[pl.ds(ebn, _CH)], src_v[b],
                                      isem[b]).wait()
                pltpu.make_async_copy(dst.at[pl.ds(ebn, _CH)], dst_v[b],
                                      isem[b]).wait()
                pltpu.async_copy(sbh.at[src_v[b]], sb_v[b], gsem_sb[b])
                pltpu.async_copy(th.at[dst_v[b]], t_v[b], gsem_t[b])
            elif pf:
                @pl.when(i < _FS - 1)
                def _():
                    pltpu.async_copy(dst.at[pl.ds(ebn, _CH)], dst_v[b],
                                     isem[b])
                    pltpu.make_async_copy(src.at[pl.ds(ebn, _CH)], src_v[b],
                                          isem[b]).wait()
                    pltpu.make_async_copy(dst.at[pl.ds(ebn, _CH)], dst_v[b],
                                          isem[b]).wait()
                    pltpu.async_copy(sbh.at[src_v[b]], sb_v[b], gsem_sb[b])
                    pltpu.async_copy(th.at[dst_v[b]], t_v[b], gsem_t[b])
            return carry

        def step(i, carry):
            for b in range(_NSLOT):
                eb = ebase + (_NSLOT * i + b) * _CH
                ebn = eb + _NSLOT * _CH
                carry = chunk_work(b, eb, ebn, carry, i, True)
            return carry

        z16 = jnp.zeros((16,), jnp.float32)
        res = lax.fori_loop(0, _FS, step, (z16,) * 8)
        for b in range(_LEFT):
            eb = ebase + (_NSLOT * _FS + b) * _CH
            res = chunk_work(b, eb, eb, res, jnp.int32(_FS), False)
        pltpu.make_async_copy(
            e_v, e_out.at[pl.ds(ebase, _CH), ecol], esem).wait()
        for cc in range(_DG // 16):
            stats_v[0, pl.ds(cc * 16, 16)] = res[cc]
            stats_v[1, pl.ds(cc * 16, 16)] = res[4 + cc]
        pltpu.sync_copy(stats_v, stats_out.at[g, pl.ds(2 * s, 2)])
        plsc.subcore_barrier()
        pltpu.sync_copy(accC.at[pl.ds(row0, _NPS), pl.ds(0, _DG)],
                        ssh_out.at[pl.ds(row0, _NPS), ecol])
        pltpu.sync_copy(accC.at[pl.ds(row0, _NPS), pl.ds(_DG, _DG)],
                        ss_out.at[pl.ds(row0, _NPS), ecol])
        plsc.subcore_barrier()

    @pl.when(c == 0)
    def _():
        process(0, sb0, t0)
        process(1, sb1, t1)

    @pl.when(c == 1)
    def _():
        process(2, sb2, t2)
        process(3, sb3, t3)


_EB = 2000


def _enorm_body(e, scale, shift, out):
    out[...] = jnp.maximum(e[...] * scale[...] + shift[...], 0.0)


def _e_norm(e_raw, scale, shift):
    return pl.pallas_call(
        _enorm_body,
        grid=(_E // _EB,),
        out_shape=jax.ShapeDtypeStruct((_E, _D), jnp.float32),
        in_specs=[
            pl.BlockSpec((_EB, _D), lambda i: (i, 0)),
            pl.BlockSpec((1, _D), lambda i: (0, 0)),
            pl.BlockSpec((1, _D), lambda i: (0, 0)),
        ],
        out_specs=pl.BlockSpec((_EB, _D), lambda i: (i, 0)),
    )(e_raw, scale, shift)


def _node_update_body(ah, ssh, ss, sn, hn, sums):
    i = pl.program_id(0)
    h = (ah[...] + ssh[...] / (ss[...] + 1e-6)) * sn[...]
    hn[...] = h
    s1 = jnp.sum(h, axis=0)[None, :]
    s2 = jnp.sum(h * h, axis=0)[None, :]
    blk = jnp.concatenate([s1, s2, jnp.zeros((6, _D), jnp.float32)], axis=0)

    @pl.when(i == 0)
    def _():
        sums[...] = blk

    @pl.when(i > 0)
    def _():
        sums[...] += blk


def _node_update(Ah, ssh, ss, snorm_n):
    return pl.pallas_call(
        _node_update_body,
        grid=(_N // _NB,),
        out_shape=[
            jax.ShapeDtypeStruct((_N, _D), jnp.float32),
            jax.ShapeDtypeStruct((8, _D), jnp.float32),
        ],
        in_specs=[
            pl.BlockSpec((_NB, _D), lambda i: (i, 0)),
            pl.BlockSpec((_NB, _D), lambda i: (i, 0)),
            pl.BlockSpec((_NB, _D), lambda i: (i, 0)),
            pl.BlockSpec((_NB, 1), lambda i: (i, 0)),
        ],
        out_specs=[
            pl.BlockSpec((_NB, _D), lambda i: (i, 0)),
            pl.BlockSpec((8, _D), lambda i: (0, 0)),
        ],
    )(Ah, ssh, ss, snorm_n)


def _hnorm_body(h, scale, shift, out):
    out[...] = jnp.maximum(h[...] * scale[...] + shift[...], 0.0)


def _h_norm(hn_pre, scale, shift):
    return pl.pallas_call(
        _hnorm_body,
        grid=(_N // _NB,),
        out_shape=jax.ShapeDtypeStruct((_N, _D), jnp.float32),
        in_specs=[
            pl.BlockSpec((_NB, _D), lambda i: (i, 0)),
            pl.BlockSpec((1, _D), lambda i: (0, 0)),
            pl.BlockSpec((1, _D), lambda i: (0, 0)),
        ],
        out_specs=pl.BlockSpec((_NB, _D), lambda i: (i, 0)),
    )(hn_pre, scale, shift)


def kernel(h, edge_index, snorm_n, W_w, W_b, V_w, V_b, A_w, A_b, B_w, B_b,
           C_w, C_b, D_w, D_b, E_w, E_b, gamma_h, beta_h, gamma_e, beta_e):
    r2 = lambda v: v.reshape(1, _D)
    Wall, ball = _prep_weights(V_w, C_w, A_w, B_w, D_w, E_w, r2(V_b),
                               r2(A_b), r2(B_b), r2(C_b), r2(D_b), r2(E_b))
    Ah, sb4, t4 = _node_matmuls(h, W_w, r2(W_b), Wall, ball)

    src = edge_index[0]
    dst = edge_index[1]
    zrows = jnp.zeros((_N, 2 * _DG), jnp.float32)
    e_raw, ssh, ss, stats = _edge_call(
        sb4[0], sb4[1], sb4[2], sb4[3], t4[0], t4[1], t4[2], t4[3],
        src, dst, zrows)

    sum_e = jnp.sum(stats[:, 0::2, :], axis=1).reshape(_D)
    sumsq_e = jnp.sum(stats[:, 1::2, :], axis=1).reshape(_D)
    mean_e = sum_e / _E
    var_e = sumsq_e / _E - mean_e * mean_e
    scale_e = gamma_e / jnp.sqrt(var_e + 1e-5)
    shift_e = beta_e - mean_e * scale_e
    e_out = _e_norm(e_raw, r2(scale_e), r2(shift_e))

    hn_pre, sums = _node_update(Ah, ssh, ss, snorm_n)
    mean_h = sums[0] / _N
    var_h = sums[1] / _N - mean_h * mean_h
    scale_h = gamma_h / jnp.sqrt(var_h + 1e-5)
    shift_h = beta_h - mean_h * scale_h
    hn = _h_norm(hn_pre, r2(scale_h), r2(shift_h))
    return (hn, e_out)
